# baseline (speedup 1.0000x reference)
import jax
import jax.numpy as jnp
from jax.experimental import pallas as pl
from jax.experimental.pallas import tpu as pltpu

_CHUNK = 256
_DEPTH = 8
_OD = 2


def _router_pipeline(x_hbm, w_ref, logits_hbm, mask_hbm,
                     xbuf, lbuf, mbuf, insem, lsem, msem):
    n_chunks = x_hbm.shape[0] // _CHUNK

    def in_copy(c, slot):
        return pltpu.make_async_copy(
            x_hbm.at[pl.ds(c * _CHUNK, _CHUNK), :], xbuf.at[slot], insem.at[slot])

    for j in range(_DEPTH):
        in_copy(j, j).start()

    def body(i, _):
        slot = jax.lax.rem(i, _DEPTH)
        oslot = jax.lax.rem(i, _OD)
        in_copy(i, slot).wait()

        @pl.when(i >= _OD)
        def _():
            pltpu.make_async_copy(
                lbuf.at[oslot],
                logits_hbm.at[pl.ds((i - _OD) * _CHUNK, _CHUNK), :],
                lsem.at[oslot]).wait()
            pltpu.make_async_copy(
                mbuf.at[oslot],
                mask_hbm.at[pl.ds((i - _OD) * _CHUNK, _CHUNK), :],
                msem.at[oslot]).wait()

        logits = jax.lax.dot_general(
            xbuf[slot],
            w_ref[...],
            dimension_numbers=(((1,), (1,)), ((), ())),
            preferred_element_type=jnp.float32,
        )
        lbuf[oslot] = logits
        mbuf[oslot] = (logits > 0).astype(jnp.int8)

        pltpu.make_async_copy(
            lbuf.at[oslot],
            logits_hbm.at[pl.ds(i * _CHUNK, _CHUNK), :],
            lsem.at[oslot]).start()
        pltpu.make_async_copy(
            mbuf.at[oslot],
            mask_hbm.at[pl.ds(i * _CHUNK, _CHUNK), :],
            msem.at[oslot]).start()

        @pl.when(i + _DEPTH < n_chunks)
        def _():
            in_copy(i + _DEPTH, slot).start()

        return 0

    jax.lax.fori_loop(0, n_chunks, body, 0)

    for t in range(_OD):
        c = n_chunks - _OD + t
        oslot = c % _OD
        pltpu.make_async_copy(
            lbuf.at[oslot],
            logits_hbm.at[pl.ds(c * _CHUNK, _CHUNK), :],
            lsem.at[oslot]).wait()
        pltpu.make_async_copy(
            mbuf.at[oslot],
            mask_hbm.at[pl.ds(c * _CHUNK, _CHUNK), :],
            msem.at[oslot]).wait()


def kernel(x, weight):
    m, k = x.shape
    e = weight.shape[0]
    logits, mask = pl.pallas_call(
        _router_pipeline,
        in_specs=[
            pl.BlockSpec(memory_space=pl.ANY),
            pl.BlockSpec(memory_space=pltpu.VMEM),
        ],
        out_specs=[
            pl.BlockSpec(memory_space=pl.ANY),
            pl.BlockSpec(memory_space=pl.ANY),
        ],
        out_shape=[
            jax.ShapeDtypeStruct((m, e), jnp.float32),
            jax.ShapeDtypeStruct((m, e), jnp.int8),
        ],
        scratch_shapes=[
            pltpu.VMEM((_DEPTH, _CHUNK, k), jnp.float32),
            pltpu.VMEM((_OD, _CHUNK, e), jnp.float32),
            pltpu.VMEM((_OD, _CHUNK, e), jnp.int8),
            pltpu.SemaphoreType.DMA((_DEPTH,)),
            pltpu.SemaphoreType.DMA((_OD,)),
            pltpu.SemaphoreType.DMA((_OD,)),
        ],
    )(x, weight)
    return (logits, mask.astype(jnp.bool_))
